# Initial kernel scaffold; baseline (speedup 1.0000x reference)
import jax
import jax.numpy as jnp
from jax.experimental import pallas as pl


def kernel(x, z, node_type, edge_index, edge_attr, iso_type_3, assignment_index_3, edge_index_3, node_to_subgraph, assignment3_to_subgraph, subgraph_to_graph, z_emb_table, nt_emb_table, nn1_W1, nn1_b1, nn1_W2, nn1_b2, conv1_root, conv1_bias, nn2_W1, nn2_b1, nn2_W2, nn2_b2, conv2_root, conv2_bias, nn3_W1, nn3_b1, nn3_W2, nn3_b2, conv3_root, conv3_bias, conv6_Wrel, conv6_brel, conv6_Wroot, conv7_Wrel, conv7_brel, conv7_Wroot, fc1_W, fc1_b, fc2_W, fc2_b, fc3_W, fc3_b):
    raise NotImplementedError("write your pallas kernel here")


# speedup vs baseline: 1.5605x; 1.5605x over previous
import functools

import jax
import jax.numpy as jnp
from jax import lax
from jax.experimental import pallas as pl
from jax.experimental.pallas import tpu as pltpu
from jax.experimental.pallas import tpu_sc as plsc

_NW = 32
_RC = 128
_BSC = _NW * _RC


def _ceil_to(x, m):
    return (x + m - 1) // m * m


def _divisor_leq(n, cap):
    for d in range(min(n, cap), 0, -1):
        if n % d == 0:
            return d
    return 1


def _row_block(n, cap=1024):
    for c in (1024, 1000, 512, 500, 256, 250, 200, 128, 125, 100, 64, 50,
              40, 32, 25, 20, 16, 10, 8, 5, 4, 2, 1):
        if c <= cap and n % c == 0:
            return c
    return 1


def _sc_gather(table, idx):
    V, D = table.shape
    B = idx.shape[0]
    cpt = B // _NW // _RC
    g = _divisor_leq(cpt, 8)
    ng = cpt // g
    idx2 = idx.reshape(_NW, cpt, _RC)
    mesh = plsc.VectorSubcoreMesh(core_axis_name="c", subcore_axis_name="s")

    @functools.partial(
        pl.kernel, mesh=mesh,
        out_type=jax.ShapeDtypeStruct((B, D), jnp.float32),
        compiler_params=pltpu.CompilerParams(use_tc_tiling_on_sc=False),
        scratch_types=[
            pltpu.VMEM((cpt, _RC), jnp.int32),
            pltpu.VMEM((g * _RC, D), jnp.float32),
            pltpu.SemaphoreType.DMA,
        ],
    )
    def k(table_hbm, idx_hbm, out_hbm, idx_v, rows_v, sem):
        w = lax.axis_index("s") * 2 + lax.axis_index("c")
        c0 = w * cpt
        pltpu.sync_copy(idx_hbm.at[w], idx_v)

        def body(i, carry):
            copies = [
                pltpu.async_copy(table_hbm.at[idx_v.at[i * g + j]],
                                 rows_v.at[pl.ds(j * _RC, _RC)], sem)
                for j in range(g)
            ]
            for cp in copies:
                cp.wait()
            pltpu.sync_copy(rows_v,
                            out_hbm.at[pl.ds((c0 + i * g) * _RC, g * _RC)])
            return carry

        lax.fori_loop(0, ng, body, 0)

    return k(table, idx2)


def _sc_scatter_add(vals, idx, racc):
    B, D = vals.shape
    cpt = B // _NW // _RC
    g = _divisor_leq(cpt, 8)
    while g > 1 and racc * D + 16 * (g * _RC * D + cpt * _RC) > 1_900_000:
        g = _divisor_leq(cpt, g - 1)
    ng = cpt // g
    idx2 = idx.reshape(_NW, cpt, _RC)
    zeros = jnp.zeros((racc, D), jnp.float32)
    rpt = racc // 16
    mesh = plsc.VectorSubcoreMesh(core_axis_name="c", subcore_axis_name="s")

    @functools.partial(
        pl.kernel, mesh=mesh,
        out_type=jax.ShapeDtypeStruct((2, racc, D), jnp.float32),
        compiler_params=pltpu.CompilerParams(use_tc_tiling_on_sc=False),
        scratch_types=[
            pltpu.VMEM((cpt, _RC), jnp.int32),
            pltpu.VMEM((g * _RC, D), jnp.float32),
            pltpu.VMEM_SHARED((racc, D), jnp.float32),
        ],
    )
    def k(vals_hbm, idx_hbm, zeros_hbm, out_hbm, idx_v, vals_v, acc_sh):
        c = lax.axis_index("c")
        s = lax.axis_index("s")
        w = s * 2 + c
        pltpu.sync_copy(zeros_hbm.at[pl.ds(s * rpt, rpt)],
                        acc_sh.at[pl.ds(s * rpt, rpt)])
        plsc.subcore_barrier()
        c0 = w * cpt
        pltpu.sync_copy(idx_hbm.at[w], idx_v)

        def body(i, carry):
            pltpu.sync_copy(vals_hbm.at[pl.ds((c0 + i * g) * _RC, g * _RC)],
                            vals_v)
            for j in range(g):
                pltpu.sync_copy(vals_v.at[pl.ds(j * _RC, _RC)],
                                acc_sh.at[idx_v.at[i * g + j]], add=True)
            return carry

        lax.fori_loop(0, ng, body, 0)
        plsc.subcore_barrier()
        pltpu.sync_copy(acc_sh.at[pl.ds(s * rpt, rpt)],
                        out_hbm.at[c].at[pl.ds(s * rpt, rpt)])

    return k(vals, idx2, zeros)


def _elu(y):
    return jnp.where(y > 0.0, y, jnp.exp(jnp.minimum(y, 0.0)) - 1.0)


def _full(shape):
    return pl.BlockSpec(shape, lambda *i: tuple(0 for _ in shape))


def _embed(gz, gnt, x, s1, s2):
    N = x.shape[0]
    WO = s1.shape[1]
    br = _row_block(N)

    def body(gz_ref, gnt_ref, x_ref, s1_ref, s2_ref, o_ref):
        o_ref[...] = (
            jnp.dot(gz_ref[...] + gnt_ref[...], s1_ref[...],
                    preferred_element_type=jnp.float32)
            + jnp.dot(x_ref[...], s2_ref[...],
                      preferred_element_type=jnp.float32))

    return pl.pallas_call(
        body,
        grid=(N // br,),
        in_specs=[
            pl.BlockSpec((br, gz.shape[1]), lambda i: (i, 0)),
            pl.BlockSpec((br, gnt.shape[1]), lambda i: (i, 0)),
            pl.BlockSpec((br, x.shape[1]), lambda i: (i, 0)),
            _full(s1.shape),
            _full(s2.shape),
        ],
        out_specs=pl.BlockSpec((br, WO), lambda i: (i, 0)),
        out_shape=jax.ShapeDtypeStruct((N, WO), jnp.float32),
    )(gz, gnt, x, s1, s2)


def _nnconv_msgs(x_src, ea, w1, b1, w2r, b2r, be):
    EP, in_pad = x_src.shape
    out = w2r.shape[1]

    def body(x_ref, e_ref, w1_ref, b1_ref, w2_ref, b2_ref, o_ref):
        h = jnp.maximum(
            jnp.dot(e_ref[...], w1_ref[...],
                    preferred_element_type=jnp.float32) + b1_ref[...], 0.0)
        xb = x_ref[...]
        p = (xb[:, :, None] * h[:, None, :]).reshape(be, in_pad * 128)
        o_ref[...] = (
            jnp.dot(p, w2_ref[...], preferred_element_type=jnp.float32)
            + jnp.dot(xb, b2_ref[...], preferred_element_type=jnp.float32))

    return pl.pallas_call(
        body,
        grid=(EP // be,),
        in_specs=[
            pl.BlockSpec((be, in_pad), lambda i: (i, 0)),
            pl.BlockSpec((be, ea.shape[1]), lambda i: (i, 0)),
            _full(w1.shape),
            _full((1, 128)),
            _full(w2r.shape),
            _full(b2r.shape),
        ],
        out_specs=pl.BlockSpec((be, out), lambda i: (i, 0)),
        out_shape=jax.ShapeDtypeStruct((EP, out), jnp.float32),
    )(x_src, ea, w1, b1.reshape(1, 128), w2r, b2r)


def _combine_nnconv(parts, h, rootp, bias, sel, cvec):
    N = h.shape[0]
    outc = rootp.shape[1]
    WO = sel.shape[1]
    br = _row_block(N)

    def body(p_ref, h_ref, r_ref, b_ref, s_ref, c_ref, o_ref):
        y = (p_ref[0] + p_ref[1]
             + jnp.dot(h_ref[...], r_ref[...],
                       preferred_element_type=jnp.float32) + b_ref[...])
        o_ref[...] = jnp.dot(_elu(y), s_ref[...],
                             preferred_element_type=jnp.float32) + c_ref[...]

    return pl.pallas_call(
        body,
        grid=(N // br,),
        in_specs=[
            pl.BlockSpec((2, br, outc), lambda i: (0, i, 0)),
            pl.BlockSpec((br, h.shape[1]), lambda i: (i, 0)),
            _full(rootp.shape),
            _full((1, outc)),
            _full(sel.shape),
            _full((1, WO)),
        ],
        out_specs=pl.BlockSpec((br, WO), lambda i: (i, 0)),
        out_shape=jax.ShapeDtypeStruct((N, WO), jnp.float32),
    )(parts, h, rootp, bias.reshape(1, outc), sel, cvec)


def _combine_gconv(parts, h, wrelp, brel, wrootp, sel, cvec):
    N, Din = h.shape
    outc = wrelp.shape[1]
    WO = sel.shape[1]
    br = _row_block(N)

    def body(p_ref, h_ref, wr_ref, b_ref, wt_ref, s_ref, c_ref, o_ref):
        agg = p_ref[0] + p_ref[1]
        y = (jnp.dot(agg, wr_ref[...], preferred_element_type=jnp.float32)
             + b_ref[...]
             + jnp.dot(h_ref[...], wt_ref[...],
                       preferred_element_type=jnp.float32))
        o_ref[...] = jnp.dot(_elu(y), s_ref[...],
                             preferred_element_type=jnp.float32) + c_ref[...]

    return pl.pallas_call(
        body,
        grid=(N // br,),
        in_specs=[
            pl.BlockSpec((2, br, Din), lambda i: (0, i, 0)),
            pl.BlockSpec((br, Din), lambda i: (i, 0)),
            _full(wrelp.shape),
            _full((1, outc)),
            _full(wrootp.shape),
            _full(sel.shape),
            _full((1, WO)),
        ],
        out_specs=pl.BlockSpec((br, WO), lambda i: (i, 0)),
        out_shape=jax.ShapeDtypeStruct((N, WO), jnp.float32),
    )(parts, h, wrelp, brel.reshape(1, outc), wrootp, sel, cvec)


def _mean_iso(parts, iso2d, smean, cmean, asel, bsel):
    N3 = iso2d.shape[0]
    D = parts.shape[2]
    WO = asel.shape[1]
    KI = bsel.shape[0]
    br = _row_block(N3)

    def body(p_ref, i_ref, s_ref, c_ref, a_ref, b_ref, o_ref):
        ps = p_ref[0] + p_ref[1]
        num = jnp.dot(ps, s_ref[...], preferred_element_type=jnp.float32)
        den = jnp.dot(ps, c_ref[...], preferred_element_type=jnp.float32)
        mean = num / jnp.maximum(den, 1.0)
        oh = (i_ref[...] == lax.broadcasted_iota(jnp.int32, (br, KI), 1))
        o_ref[...] = (
            jnp.dot(mean, a_ref[...], preferred_element_type=jnp.float32)
            + jnp.dot(oh.astype(jnp.float32), b_ref[...],
                      preferred_element_type=jnp.float32))

    return pl.pallas_call(
        body,
        grid=(N3 // br,),
        in_specs=[
            pl.BlockSpec((2, br, D), lambda i: (0, i, 0)),
            pl.BlockSpec((br, 1), lambda i: (i, 0)),
            _full(smean.shape),
            _full(cmean.shape),
            _full(asel.shape),
            _full(bsel.shape),
        ],
        out_specs=pl.BlockSpec((br, WO), lambda i: (i, 0)),
        out_shape=jax.ShapeDtypeStruct((N3, WO), jnp.float32),
    )(parts, iso2d, smean, cmean, asel, bsel)


def _pool_combine(p1, p3, smean, cmean, a1, a3, ones_c):
    R = p1.shape[1]
    D = p1.shape[2]
    WO = a1.shape[1]

    def body(p1_ref, p3_ref, s_ref, c_ref, a1_ref, a3_ref, o_ref, out_ref):
        s1 = p1_ref[0] + p1_ref[1]
        s3 = p3_ref[0] + p3_ref[1]
        x1 = (jnp.dot(s1, s_ref[...], preferred_element_type=jnp.float32)
              / jnp.maximum(jnp.dot(s1, c_ref[...],
                                    preferred_element_type=jnp.float32), 1.0))
        x3 = (jnp.dot(s3, s_ref[...], preferred_element_type=jnp.float32)
              / jnp.maximum(jnp.dot(s3, c_ref[...],
                                    preferred_element_type=jnp.float32), 1.0))
        out_ref[...] = (
            jnp.dot(x1, a1_ref[...], preferred_element_type=jnp.float32)
            + jnp.dot(x3, a3_ref[...], preferred_element_type=jnp.float32)
            + o_ref[...])

    return pl.pallas_call(
        body,
        in_specs=[
            _full((2, R, D)),
            _full((2, R, D)),
            _full(smean.shape),
            _full(cmean.shape),
            _full(a1.shape),
            _full(a3.shape),
            _full((1, WO)),
        ],
        out_specs=_full((R, WO)),
        out_shape=jax.ShapeDtypeStruct((R, WO), jnp.float32),
    )(p1, p3, smean, cmean, a1, a3, ones_c)


def _final_mlp(parts, smean, cmean, w1, b1, w2, b2, w3, b3):
    R = parts.shape[1]
    D = parts.shape[2]
    WO = w3.shape[1]

    def body(p_ref, s_ref, c_ref, w1_ref, b1_ref, w2_ref, b2_ref, w3_ref,
             b3_ref, o_ref):
        ps = p_ref[0] + p_ref[1]
        hg = (jnp.dot(ps, s_ref[...], preferred_element_type=jnp.float32)
              / jnp.maximum(jnp.dot(ps, c_ref[...],
                                    preferred_element_type=jnp.float32), 1.0))
        h1 = _elu(jnp.dot(hg, w1_ref[...],
                          preferred_element_type=jnp.float32) + b1_ref[...])
        h2 = _elu(jnp.dot(h1, w2_ref[...],
                          preferred_element_type=jnp.float32) + b2_ref[...])
        o_ref[...] = jnp.dot(h2, w3_ref[...],
                             preferred_element_type=jnp.float32) + b3_ref[...]

    return pl.pallas_call(
        body,
        in_specs=[
            _full((2, R, D)),
            _full(smean.shape),
            _full(cmean.shape),
            _full(w1.shape),
            _full((1, w1.shape[1])),
            _full(w2.shape),
            _full((1, w2.shape[1])),
            _full(w3.shape),
            _full((1, WO)),
        ],
        out_specs=_full((R, WO)),
        out_shape=jax.ShapeDtypeStruct((R, WO), jnp.float32),
    )(parts, smean, cmean, w1, b1, w2, b2, w3, b3)


def _eye_pad(n_in, n_out, row0=0, col0=0, n=None):
    n = min(n_in - row0, n_out - col0) if n is None else n
    m = jnp.zeros((n_in, n_out), jnp.float32)
    return m.at[jnp.arange(n) + row0, jnp.arange(n) + col0].set(1.0)


def _onehot_row(n_in, n_out, row, val=1.0):
    m = jnp.zeros((n_in, n_out), jnp.float32)
    return m.at[row, :].set(val)


def _pad_rows(a, rows, value=0):
    return jnp.pad(a, ((0, rows - a.shape[0]),) + ((0, 0),) * (a.ndim - 1),
                   constant_values=value)


def _prep_w2(w2, b2, in_real, in_pad, out):
    w2r = w2.reshape(128, in_real, out).transpose(1, 0, 2)
    w2r = jnp.pad(w2r, ((0, in_pad - in_real), (0, 0), (0, 0)))
    b2r = jnp.pad(b2.reshape(in_real, out), ((0, in_pad - in_real), (0, 0)))
    return w2r.reshape(in_pad * 128, out), b2r


def kernel(x, z, node_type, edge_index, edge_attr, iso_type_3,
           assignment_index_3, edge_index_3, node_to_subgraph,
           assignment3_to_subgraph, subgraph_to_graph, z_emb_table,
           nt_emb_table, nn1_W1, nn1_b1, nn1_W2, nn1_b2, conv1_root,
           conv1_bias, nn2_W1, nn2_b1, nn2_W2, nn2_b2, conv2_root, conv2_bias,
           nn3_W1, nn3_b1, nn3_W2, nn3_b2, conv3_root, conv3_bias, conv6_Wrel,
           conv6_brel, conv6_Wroot, conv7_Wrel, conv7_brel, conv7_Wroot,
           fc1_W, fc1_b, fc2_W, fc2_b, fc3_W, fc3_b):
    i32 = jnp.int32
    N, NF = x.shape
    E = edge_index.shape[1]
    N3 = iso_type_3.shape[0]
    A = assignment_index_3.shape[1]
    E3 = edge_index_3.shape[1]
    S = subgraph_to_graph.shape[0]
    G = 64
    KI = conv6_Wrel.shape[0] - 64
    KIP = _ceil_to(KI, 16)

    EP = _ceil_to(E, _BSC)
    E3P = _ceil_to(E3, _BSC)
    AP = _ceil_to(A, _BSC)
    NPd = _ceil_to(N, _BSC)
    N3P = _ceil_to(N3, _BSC)
    SP = _ceil_to(S, _BSC)
    RN = _ceil_to(N + 1, 128)
    RN3 = _ceil_to(N3 + 1, 128)
    RS = _ceil_to(S + 1, 128)
    RG = _ceil_to(G + 1, 128)

    src = edge_index[0].astype(i32)
    dst = edge_index[1].astype(i32)
    src_p = _pad_rows(src.reshape(-1, 1), EP).reshape(-1)
    dst_p = _pad_rows(dst.reshape(-1, 1), EP, value=N).reshape(-1)
    src3_p = _pad_rows(edge_index_3[0].astype(i32).reshape(-1, 1),
                       E3P).reshape(-1)
    dst3_p = _pad_rows(edge_index_3[1].astype(i32).reshape(-1, 1), E3P,
                       value=N3).reshape(-1)
    row_p = _pad_rows(assignment_index_3[0].astype(i32).reshape(-1, 1),
                      AP).reshape(-1)
    col_p = _pad_rows(assignment_index_3[1].astype(i32).reshape(-1, 1), AP,
                      value=N3).reshape(-1)
    n2s_p = _pad_rows(node_to_subgraph.astype(i32).reshape(-1, 1), NPd,
                      value=S).reshape(-1)
    a3s_p = _pad_rows(assignment3_to_subgraph.astype(i32).reshape(-1, 1), N3P,
                      value=S).reshape(-1)
    s2g_p = _pad_rows(subgraph_to_graph.astype(i32).reshape(-1, 1), SP,
                      value=G).reshape(-1)

    ea_p = _pad_rows(edge_attr, EP)

    IN1 = NF + 8
    IN1P = _ceil_to(IN1, 32)
    zt = jnp.pad(z_emb_table, ((0, 0), (0, 8)))
    ntt = jnp.pad(nt_emb_table, ((0, 0), (0, 8)))
    z_pi = _pad_rows(z.astype(i32).reshape(-1, 1), NPd).reshape(-1)
    nt_pi = _pad_rows(node_type.astype(i32).reshape(-1, 1), NPd).reshape(-1)
    gz = _sc_gather(zt, z_pi)[:N]
    gnt = _sc_gather(ntt, nt_pi)[:N]
    s1 = _eye_pad(16, IN1P, n=8)
    s2 = _eye_pad(NF, IN1P, col0=8)
    h0 = _embed(gz, gnt, x, s1, s2)

    def nnconv(h, in_real, out_ch, w1, b1, w2, b2, root, bias, sel, cvec, be):
        in_pad = h.shape[1]
        xs = _sc_gather(h, src_p)
        w2r, b2r = _prep_w2(w2, b2, in_real, in_pad, out_ch)
        msgs = _nnconv_msgs(xs, ea_p, w1, b1, w2r, b2r, be)
        parts = _sc_scatter_add(msgs, dst_p, RN)
        rootp = _pad_rows(root, in_pad)
        return _combine_nnconv(parts, h, rootp, bias, sel, cvec)

    h1 = nnconv(h0, IN1, 32, nn1_W1, nn1_b1, nn1_W2, nn1_b2, conv1_root,
                conv1_bias, _eye_pad(32, 32), jnp.zeros((1, 32), jnp.float32),
                512)
    h2 = nnconv(h1, 32, 64, nn2_W1, nn2_b1, nn2_W2, nn2_b2, conv2_root,
                conv2_bias, _eye_pad(64, 64), jnp.zeros((1, 64), jnp.float32),
                512)
    h3x = nnconv(h2, 64, 64, nn3_W1, nn3_b1, nn3_W2, nn3_b2, conv3_root,
                 conv3_bias, _eye_pad(64, 80), _eye_pad(1, 80, col0=64), 256)

    parts_x1 = _sc_scatter_add(_pad_rows(h3x, NPd), n2s_p, RS)

    hr = _sc_gather(h3x, row_p)
    parts_h3 = _sc_scatter_add(hr, col_p, RN3)
    sm80 = _eye_pad(80, 64)
    cm80 = _onehot_row(80, 64, 64)
    h3m = _mean_iso(parts_h3, iso_type_3.astype(i32).reshape(-1, 1), sm80,
                    cm80, _eye_pad(64, 80), _eye_pad(KIP, 80, col0=64, n=KI))

    def gconv(hh, wrel, brel, wroot, sel, cvec):
        din = hh.shape[1]
        xs = _sc_gather(hh, src3_p)
        parts = _sc_scatter_add(xs, dst3_p, RN3)
        wrelp = _pad_rows(wrel, din)
        wrootp = _pad_rows(wroot, din)
        return _combine_gconv(parts, hh, wrelp, brel, wrootp, sel, cvec)

    h6 = gconv(h3m, conv6_Wrel, conv6_brel, conv6_Wroot, _eye_pad(64, 64),
               jnp.zeros((1, 64), jnp.float32))
    h7x = gconv(h6, conv7_Wrel, conv7_brel, conv7_Wroot, _eye_pad(64, 80),
                _eye_pad(1, 80, col0=64))

    parts_x3 = _sc_scatter_add(_pad_rows(h7x, N3P), a3s_p, RS)
    hs = _pool_combine(parts_x1, parts_x3, sm80, cm80, _eye_pad(64, 144),
                       _eye_pad(64, 144, col0=64),
                       _eye_pad(1, 144, col0=128).reshape(1, 144))

    hs_p = _pad_rows(hs[:S], SP)
    parts_hg = _sc_scatter_add(hs_p, s2g_p, RG)
    out = _final_mlp(parts_hg, _eye_pad(144, 128), _onehot_row(144, 128, 128),
                     fc1_W, fc1_b.reshape(1, -1), fc2_W, fc2_b.reshape(1, -1),
                     jnp.pad(fc3_W, ((0, 0), (0, 7))),
                     jnp.pad(fc3_b.reshape(1, -1), ((0, 0), (0, 7))))
    return out[:G, 0].reshape(-1)
